# baseline (speedup 1.0000x reference)
import functools

import jax
import jax.numpy as jnp
from jax import lax
from jax.experimental import pallas as pl
from jax.experimental.pallas import tpu as pltpu
from jax.experimental.pallas import tpu_sc as plsc

PAD_LEN = 20
EMBED_DIM = 50
GROUP = 16


@functools.lru_cache(maxsize=None)
def _build(B, V):
    info = plsc.get_sparse_core_info()
    NC, NS = info.num_cores, info.num_subcores
    NW = NC * NS
    e_per_w = B // NW
    n_grp = e_per_w // GROUP
    P, D = PAD_LEN, EMBED_DIM
    mesh = plsc.VectorSubcoreMesh(core_axis_name="c", subcore_axis_name="s")

    @functools.partial(
        pl.kernel,
        mesh=mesh,
        out_type=jax.ShapeDtypeStruct((B, P, D), jnp.float32),
        scratch_types=[
            pltpu.VMEM((e_per_w * P,), jnp.int32),
            pltpu.VMEM((e_per_w + 8,), jnp.int32),
            pltpu.VMEM((2, GROUP, P, D), jnp.float32),
            pltpu.SemaphoreType.DMA,
            pltpu.SemaphoreType.DMA,
            pltpu.SemaphoreType.DMA,
        ],
    )
    def k(idx_hbm, len_hbm, table_hbm, out_hbm, idx_v, len_v, stage, isem, gsem, wsem):
        wid = lax.axis_index("s") * NC + lax.axis_index("c")
        ebase = wid * e_per_w
        pltpu.async_copy(idx_hbm.at[wid], idx_v, isem).wait()
        pltpu.async_copy(len_hbm.at[wid], len_v.at[pl.ds(0, e_per_w)], isem).wait()
        zvec = jnp.zeros((16,), jnp.float32)

        def fire_group(g, b):
            lnv = len_v[pl.ds(g * GROUP, 16)]
            for e in range(GROUP):
                ln = lnv[e]
                off = g * (GROUP * P) + e * P
                w0 = idx_v[pl.ds(off, 16)]
                w1 = idx_v[pl.ds(off + 4, 16)]
                for p in range(P):
                    v = w0[p] if p < 16 else w1[p - 4]

                    @pl.when(p < ln)
                    def _():
                        pltpu.make_async_copy(
                            table_hbm.at[v], stage.at[b, e, p], gsem
                        ).start()
            cnt = lnv[0]
            for e in range(1, GROUP):
                cnt = cnt + lnv[e]
            return cnt

        def drain_group(b, cnt):
            @pl.loop(0, cnt)
            def _(i):
                pltpu.make_async_copy(
                    table_hbm.at[0], stage.at[b, 0, 0], gsem
                ).wait()

        def mask_group(g, b):
            lnv = len_v[pl.ds(g * GROUP, 16)]
            for e in range(GROUP):
                ln = lnv[e]

                @pl.loop(ln, P)
                def _(p):
                    stage[b, e, p, pl.ds(0, 16)] = zvec
                    stage[b, e, p, pl.ds(16, 16)] = zvec
                    stage[b, e, p, pl.ds(32, 16)] = zvec
                    stage[b, e, p, pl.ds(34, 16)] = zvec

        def write_group(g, b):
            pltpu.make_async_copy(
                stage.at[b], out_hbm.at[pl.ds(ebase + g * GROUP, GROUP)], wsem
            ).start()

        def wait_write(b):
            pltpu.make_async_copy(
                stage.at[b], out_hbm.at[pl.ds(0, GROUP)], wsem
            ).wait()

        cnt0 = fire_group(0, 0)

        def loop_body(gp, cnt):
            b = lax.rem(gp, 2)
            nb = lax.rem(gp + 1, 2)

            @pl.when(gp >= 1)
            def _():
                wait_write(nb)

            ncnt = fire_group(gp + 1, nb)
            drain_group(b, cnt)
            mask_group(gp, b)
            write_group(gp, b)
            return ncnt

        cntl = lax.fori_loop(0, n_grp - 1, loop_body, cnt0)

        gl = n_grp - 1
        bl = (n_grp - 1) % 2
        wait_write(1 - bl)
        drain_group(bl, cntl)
        mask_group(gl, bl)
        write_group(gl, bl)
        wait_write(bl)

    return k


def kernel(indices, lengths, table):
    B, P = indices.shape
    V, D = table.shape
    info = plsc.get_sparse_core_info()
    NW = info.num_cores * info.num_subcores
    idx2 = indices.reshape(NW, (B // NW) * P).astype(jnp.int32)
    len2 = lengths.reshape(NW, B // NW).astype(jnp.int32)
    return _build(B, V)(idx2, len2, table)


# speedup vs baseline: 1.0250x; 1.0250x over previous
import functools

import jax
import jax.numpy as jnp
from jax import lax
from jax.experimental import pallas as pl
from jax.experimental.pallas import tpu as pltpu
from jax.experimental.pallas import tpu_sc as plsc

PAD_LEN = 20
EMBED_DIM = 50
GROUP = 16


@functools.lru_cache(maxsize=None)
def _build(B, V):
    info = plsc.get_sparse_core_info()
    NC, NS = info.num_cores, info.num_subcores
    NW = NC * NS
    e_per_w = B // NW
    n_grp = e_per_w // GROUP
    P, D = PAD_LEN, EMBED_DIM
    mesh = plsc.VectorSubcoreMesh(core_axis_name="c", subcore_axis_name="s")

    @functools.partial(
        pl.kernel,
        mesh=mesh,
        out_type=jax.ShapeDtypeStruct((B, P, D), jnp.float32),
        scratch_types=[
            pltpu.VMEM((e_per_w * P,), jnp.int32),
            pltpu.VMEM((e_per_w + 8,), jnp.int32),
            pltpu.VMEM((2, GROUP, P, D), jnp.float32),
            pltpu.SemaphoreType.DMA,
            pltpu.SemaphoreType.DMA,
            pltpu.SemaphoreType.DMA,
        ],
    )
    def k(idx_hbm, len_hbm, table_hbm, out_hbm, idx_v, len_v, stage, isem, gsem, wsem):
        wid = lax.axis_index("s") * NC + lax.axis_index("c")
        ebase = wid * e_per_w
        pltpu.async_copy(idx_hbm.at[wid], idx_v, isem).wait()
        pltpu.async_copy(len_hbm.at[wid], len_v.at[pl.ds(0, e_per_w)], isem).wait()
        zvec = jnp.zeros((16,), jnp.float32)

        def fire_group(g, b):
            for e in range(GROUP):
                off = g * (GROUP * P) + e * P
                w0 = idx_v[pl.ds(off, 16)]
                w1 = idx_v[pl.ds(off + 4, 16)]
                for p in range(P):
                    v = w0[p] if p < 16 else w1[p - 4]
                    pltpu.make_async_copy(
                        table_hbm.at[v], stage.at[b, e, p], gsem
                    ).start()
            return 0

        def drain_group(b, cnt):
            @pl.loop(0, GROUP * P)
            def _(i):
                pltpu.make_async_copy(
                    table_hbm.at[0], stage.at[b, 0, 0], gsem
                ).wait()

        def mask_group(g, b):
            lnv = len_v[pl.ds(g * GROUP, 16)]
            for e in range(GROUP):
                ln = lnv[e]

                @pl.loop(ln, P)
                def _(p):
                    stage[b, e, p, pl.ds(0, 16)] = zvec
                    stage[b, e, p, pl.ds(16, 16)] = zvec
                    stage[b, e, p, pl.ds(32, 16)] = zvec
                    stage[b, e, p, pl.ds(34, 16)] = zvec

        def write_group(g, b):
            pltpu.make_async_copy(
                stage.at[b], out_hbm.at[pl.ds(ebase + g * GROUP, GROUP)], wsem
            ).start()

        def wait_write(b):
            pltpu.make_async_copy(
                stage.at[b], out_hbm.at[pl.ds(0, GROUP)], wsem
            ).wait()

        cnt0 = fire_group(0, 0)

        def loop_body(gp, cnt):
            b = lax.rem(gp, 2)
            nb = lax.rem(gp + 1, 2)

            @pl.when(gp >= 1)
            def _():
                wait_write(nb)

            ncnt = fire_group(gp + 1, nb)
            drain_group(b, cnt)
            mask_group(gp, b)
            write_group(gp, b)
            return ncnt

        cntl = lax.fori_loop(0, n_grp - 1, loop_body, cnt0)

        gl = n_grp - 1
        bl = (n_grp - 1) % 2
        wait_write(1 - bl)
        drain_group(bl, cntl)
        mask_group(gl, bl)
        write_group(gl, bl)
        wait_write(bl)

    return k


def kernel(indices, lengths, table):
    B, P = indices.shape
    V, D = table.shape
    info = plsc.get_sparse_core_info()
    NW = info.num_cores * info.num_subcores
    idx2 = indices.reshape(NW, (B // NW) * P).astype(jnp.int32)
    len2 = lengths.reshape(NW, B // NW).astype(jnp.int32)
    return _build(B, V)(idx2, len2, table)


# speedup vs baseline: 1.0960x; 1.0693x over previous
import functools

import jax
import jax.numpy as jnp
from jax import lax
from jax.experimental import pallas as pl
from jax.experimental.pallas import tpu as pltpu
from jax.experimental.pallas import tpu_sc as plsc

PAD_LEN = 20
EMBED_DIM = 50
GROUP = 8


@functools.lru_cache(maxsize=None)
def _build(B, V):
    info = plsc.get_sparse_core_info()
    NC, NS = info.num_cores, info.num_subcores
    NW = NC * NS
    e_per_w = B // NW
    n_grp = e_per_w // GROUP
    P, D = PAD_LEN, EMBED_DIM
    mesh = plsc.VectorSubcoreMesh(core_axis_name="c", subcore_axis_name="s")

    @functools.partial(
        pl.kernel,
        mesh=mesh,
        out_type=jax.ShapeDtypeStruct((B, P, D), jnp.float32),
        scratch_types=[
            pltpu.VMEM((e_per_w * P,), jnp.int32),
            pltpu.VMEM((e_per_w + 8,), jnp.int32),
            pltpu.VMEM((2, GROUP, P, D), jnp.float32),
            pltpu.SemaphoreType.DMA,
            pltpu.SemaphoreType.DMA,
            pltpu.SemaphoreType.DMA,
        ],
    )
    def k(idx_hbm, len_hbm, table_hbm, out_hbm, idx_v, len_v, stage, isem, gsem, wsem):
        wid = lax.axis_index("s") * NC + lax.axis_index("c")
        ebase = wid * e_per_w
        pltpu.async_copy(idx_hbm.at[wid], idx_v, isem).wait()
        pltpu.async_copy(len_hbm.at[wid], len_v.at[pl.ds(0, e_per_w)], isem).wait()
        zvec = jnp.zeros((16,), jnp.float32)

        def fire_group(g, b):
            for e in range(GROUP):
                off = g * (GROUP * P) + e * P
                w0 = idx_v[pl.ds(off, 16)]
                w1 = idx_v[pl.ds(off + 4, 16)]
                for p in range(P):
                    v = w0[p] if p < 16 else w1[p - 4]
                    pltpu.make_async_copy(
                        table_hbm.at[v], stage.at[b, e, p], gsem
                    ).start()

        def drain_group(b):
            for _ in range(GROUP * P):
                pltpu.make_async_copy(
                    table_hbm.at[0], stage.at[b, 0, 0], gsem
                ).wait()

        def mask_group(g, b):
            lnv = len_v[pl.ds(g * GROUP, 16)]
            for e in range(GROUP):
                ln = lnv[e]

                @pl.loop(ln, P)
                def _(p):
                    stage[b, e, p, pl.ds(0, 16)] = zvec
                    stage[b, e, p, pl.ds(16, 16)] = zvec
                    stage[b, e, p, pl.ds(32, 16)] = zvec
                    stage[b, e, p, pl.ds(34, 16)] = zvec

        def write_group(g, b):
            pltpu.make_async_copy(
                stage.at[b], out_hbm.at[pl.ds(ebase + g * GROUP, GROUP)], wsem
            ).start()

        def wait_write(b):
            pltpu.make_async_copy(
                stage.at[b], out_hbm.at[pl.ds(0, GROUP)], wsem
            ).wait()

        fire_group(0, 0)

        @pl.loop(0, n_grp - 1)
        def _(gp):
            b = lax.rem(gp, 2)
            nb = lax.rem(gp + 1, 2)

            @pl.when(gp >= 1)
            def _():
                wait_write(nb)

            fire_group(gp + 1, nb)
            drain_group(b)
            mask_group(gp, b)
            write_group(gp, b)

        gl = n_grp - 1
        bl = (n_grp - 1) % 2
        wait_write(1 - bl)
        drain_group(bl)
        mask_group(gl, bl)
        write_group(gl, bl)
        wait_write(bl)

    return k


def kernel(indices, lengths, table):
    B, P = indices.shape
    V, D = table.shape
    info = plsc.get_sparse_core_info()
    NW = info.num_cores * info.num_subcores
    idx2 = indices.reshape(NW, (B // NW) * P).astype(jnp.int32)
    len2 = lengths.reshape(NW, B // NW).astype(jnp.int32)
    return _build(B, V)(idx2, len2, table)
